# baseline (speedup 1.0000x reference)
import functools

import jax
import jax.numpy as jnp
from jax import lax
from jax.experimental import pallas as pl
from jax.experimental.pallas import tpu as pltpu, tpu_sc as plsc

_PPW = 56
_C = 96
_W = 224


def _sc_relu(x_hbm, o_hbm, ibufs, obufs, isems, osems):
    c = lax.axis_index("c")
    s = lax.axis_index("s")
    wid = s * 2 + c
    i = wid // 8
    j = (wid // 4) % 2
    k0 = (wid % 4) * _PPW

    def in_copy(step, b):
        return pltpu.make_async_copy(
            x_hbm.at[i, j, k0 + step], ibufs[b], isems[b]
        )

    def out_copy(step, b):
        return pltpu.make_async_copy(
            obufs[b], o_hbm.at[i, j, k0 + step], osems[b]
        )

    in_copy(0, 0).start()
    in_copy(1, 1).start()

    def round_(r, _):
        for b in range(2):
            step = r * 2 + b

            @pl.when(r > 0)
            def _():
                out_copy(step - 2, b).wait()

            in_copy(step, b).wait()

            def row(q, _):
                for l in range(14):
                    obufs[b][q, pl.ds(l * 16, 16)] = jnp.maximum(
                        ibufs[b][q, pl.ds(l * 16, 16)], 0.0
                    )
                return 0

            lax.fori_loop(0, _C, row, 0, unroll=4)
            out_copy(step, b).start()

            @pl.when(step + 2 < _PPW)
            def _():
                in_copy(step + 2, b).start()

        return 0

    lax.fori_loop(0, _PPW // 2, round_, 0)
    out_copy(_PPW - 2, 0).wait()
    out_copy(_PPW - 1, 1).wait()


def kernel(inputs):
    b0, b1, h, w, c = inputs.shape
    xt = inputs.transpose(0, 1, 2, 4, 3)
    mesh = plsc.VectorSubcoreMesh(core_axis_name="c", subcore_axis_name="s")
    k = functools.partial(
        pl.kernel,
        mesh=mesh,
        out_type=jax.ShapeDtypeStruct(xt.shape, jnp.float32),
        scratch_types=[
            [pltpu.VMEM((c, w), jnp.float32) for _ in range(2)],
            [pltpu.VMEM((c, w), jnp.float32) for _ in range(2)],
            [pltpu.SemaphoreType.DMA for _ in range(2)],
            [pltpu.SemaphoreType.DMA for _ in range(2)],
        ],
    )(_sc_relu)
    out = k(xt)
    return out.transpose(0, 1, 2, 4, 3)


# speedup vs baseline: 1.3897x; 1.3897x over previous
import functools

import jax
import jax.numpy as jnp
from jax import lax
from jax.experimental import pallas as pl
from jax.experimental.pallas import tpu as pltpu, tpu_sc as plsc

_PPW = 56
_NBI = 3
_NBO = 2
_C = 96
_W = 224


def _sc_relu(x_hbm, o_hbm, ibufs, obufs, isems, osems):
    c = lax.axis_index("c")
    s = lax.axis_index("s")
    wid = s * 2 + c
    i = wid // 8
    j = (wid // 4) % 2
    k0 = (wid % 4) * _PPW

    def in_copy(step, b):
        return pltpu.make_async_copy(
            x_hbm.at[i, j, k0 + step], ibufs[b], isems[b]
        )

    def out_copy(step, b):
        return pltpu.make_async_copy(
            obufs[b], o_hbm.at[i, j, k0 + step], osems[b]
        )

    def compute(bi, bo):
        def row(q, _):
            for l in range(14):
                obufs[bo][q, pl.ds(l * 16, 16)] = jnp.maximum(
                    ibufs[bi][q, pl.ds(l * 16, 16)], 0.0
                )
            return 0

        lax.fori_loop(0, _C, row, 0, unroll=2)

    def stage(step, bi, bo):
        if isinstance(step, int):
            if step >= _NBO:
                out_copy(step - _NBO, bo).wait()
            in_copy(step, bi).wait()
            compute(bi, bo)
            out_copy(step, bo).start()
            if step + _NBI < _PPW:
                in_copy(step + _NBI, bi).start()
            return

        @pl.when(step >= _NBO)
        def _():
            out_copy(step - _NBO, bo).wait()

        in_copy(step, bi).wait()
        compute(bi, bo)
        out_copy(step, bo).start()

        @pl.when(step + _NBI < _PPW)
        def _():
            in_copy(step + _NBI, bi).start()

    for b in range(_NBI):
        in_copy(b, b).start()

    period = _NBI * _NBO
    n_main = _PPW - (_PPW % period)

    def round_(r, _):
        for u in range(period):
            stage(r * period + u, u % _NBI, u % _NBO)
        return 0

    lax.fori_loop(0, n_main // period, round_, 0)
    for t in range(n_main, _PPW):
        stage(t, t % _NBI, t % _NBO)
    for t in range(_PPW - _NBO, _PPW):
        out_copy(t, t % _NBO).wait()


def kernel(inputs):
    b0, b1, h, w, c = inputs.shape
    xt = inputs.transpose(0, 1, 2, 4, 3)
    mesh = plsc.VectorSubcoreMesh(core_axis_name="c", subcore_axis_name="s")
    k = functools.partial(
        pl.kernel,
        mesh=mesh,
        out_type=jax.ShapeDtypeStruct(xt.shape, jnp.float32),
        scratch_types=[
            [pltpu.VMEM((c, w), jnp.float32) for _ in range(_NBI)],
            [pltpu.VMEM((c, w), jnp.float32) for _ in range(_NBO)],
            [pltpu.SemaphoreType.DMA for _ in range(_NBI)],
            [pltpu.SemaphoreType.DMA for _ in range(_NBO)],
        ],
    )(_sc_relu)
    out = k(xt)
    return out.transpose(0, 1, 2, 4, 3)


# speedup vs baseline: 1.3935x; 1.0028x over previous
import functools

import jax
import jax.numpy as jnp
from jax import lax
from jax.experimental import pallas as pl
from jax.experimental.pallas import tpu as pltpu, tpu_sc as plsc

_PPW = 56
_C = 96
_W = 224


def _sc_relu(x_hbm, o_hbm, ibufs, obufs, isems, osems):
    c = lax.axis_index("c")
    s = lax.axis_index("s")
    wid = s * 2 + c
    i = wid // 8
    j = (wid // 4) % 2
    k0 = (wid % 4) * _PPW

    def in_copy(step, b):
        return pltpu.make_async_copy(
            x_hbm.at[i, j, k0 + step], ibufs[b], isems[b]
        )

    def out_copy(step, b):
        return pltpu.make_async_copy(
            obufs[b], o_hbm.at[i, j, k0 + step], osems[b]
        )

    in_copy(0, 0).start()
    in_copy(1, 1).start()

    def round_(r, _):
        for b in range(2):
            step = r * 2 + b

            @pl.when(r > 0)
            def _():
                out_copy(step - 2, b).wait()

            in_copy(step, b).wait()

            def row(q, _):
                for l in range(14):
                    obufs[b][q, pl.ds(l * 16, 16)] = jnp.maximum(
                        ibufs[b][q, pl.ds(l * 16, 16)], 0.0
                    )
                return 0

            lax.fori_loop(0, _C, row, 0, unroll=2)
            out_copy(step, b).start()

            @pl.when(step + 2 < _PPW)
            def _():
                in_copy(step + 2, b).start()

        return 0

    lax.fori_loop(0, _PPW // 2, round_, 0)
    out_copy(_PPW - 2, 0).wait()
    out_copy(_PPW - 1, 1).wait()


def kernel(inputs):
    b0, b1, h, w, c = inputs.shape
    xt = inputs.transpose(0, 1, 2, 4, 3)
    mesh = plsc.VectorSubcoreMesh(core_axis_name="c", subcore_axis_name="s")
    k = functools.partial(
        pl.kernel,
        mesh=mesh,
        out_type=jax.ShapeDtypeStruct(xt.shape, jnp.float32),
        scratch_types=[
            [pltpu.VMEM((c, w), jnp.float32) for _ in range(2)],
            [pltpu.VMEM((c, w), jnp.float32) for _ in range(2)],
            [pltpu.SemaphoreType.DMA for _ in range(2)],
            [pltpu.SemaphoreType.DMA for _ in range(2)],
        ],
    )(_sc_relu)
    out = k(xt)
    return out.transpose(0, 1, 2, 4, 3)
